# baseline (speedup 1.0000x reference)
import functools

import jax
import jax.numpy as jnp
from jax import lax
from jax.experimental import pallas as pl
from jax.experimental.pallas import tpu as pltpu
from jax.experimental.pallas import tpu_sc as plsc

NSEG = 10000
N = 320000
D = 128

_NC = 2
_CHUNK = N // _NC

_G = 64
_HG = _G // 2
_NGR = _CHUNK // _G
_NW = 32
_S = 320
_LAST = NSEG - 31 * _S
_MASK_HI = -65536
_MASK_LO = 0xFFFF


_MM_BLK = 3200


def _mm_body(x_ref, w_ref, b_ref, y_ref):
    y = lax.dot_general(
        x_ref[...].astype(jnp.bfloat16), w_ref[...].astype(jnp.bfloat16),
        (((1,), (1,)), ((), ())),
        preferred_element_type=jnp.float32)
    y = jnp.maximum(y + b_ref[...], 0.0)
    bits = lax.bitcast_convert_type(
        y.astype(jnp.bfloat16).astype(jnp.float32), jnp.int32)
    z = bits.reshape(_MM_BLK // _G, _G, D)
    a = z[:, :_HG, :]
    b = z[:, _HG:, :]
    w = ((a >> 16) & _MASK_LO) | (b & _MASK_HI)
    y_ref[...] = w.reshape(_MM_BLK // 2, D)


def _matmul_relu_chunk(x, W, b2d, k):
    nblk = _CHUNK // _MM_BLK
    return pl.pallas_call(
        _mm_body,
        grid=(nblk,),
        in_specs=[
            pl.BlockSpec((_MM_BLK, D), lambda i, k=k, nblk=nblk: (i + k * nblk, 0)),
            pl.BlockSpec((D, D), lambda i: (0, 0)),
            pl.BlockSpec((1, D), lambda i: (0, 0)),
        ],
        out_specs=pl.BlockSpec((_MM_BLK // 2, D), lambda i: (i, 0)),
        out_shape=jax.ShapeDtypeStruct((_CHUNK // 2, D), jnp.int32),
    )(x, W, b2d)


def _make_segsum_body(k):
    base_row = k * _CHUNK

    def _segsum_body(y_hbm, idx_hbm, bnd_hbm, out_hbm,
                     bv, pa, pb, ia, ib, acc, sla, slb):
        c = lax.axis_index("c")
        s = lax.axis_index("s")
        wid = c * 16 + s
        lo = wid * _S

        pltpu.sync_copy(bnd_hbm, bv)

        def _rgather(vec, lane_splat):
            return lax.gather(
                vec, lane_splat[:, None],
                lax.GatherDimensionNumbers(
                    offset_dims=(), collapsed_slice_dims=(0,),
                    start_index_map=(0,)),
                (1,), mode=lax.GatherScatterMode.PROMISE_IN_BOUNDS)

        lane = jnp.full((16,), wid & 15, jnp.int32)
        gs_v = jnp.where(wid < 16, bv[pl.ds(0, 16)], bv[pl.ds(16, 16)])
        cnt_v = jnp.where(wid < 16, bv[pl.ds(32, 16)], bv[pl.ds(48, 16)])
        gs = jnp.max(_rgather(gs_v, lane))
        cnt = jnp.max(_rgather(cnt_v, lane))

        fz = jnp.zeros((16,), jnp.float32)

        def zbody(r, carry):
            for j in range(D // 16):
                acc[r, pl.ds(16 * j, 16)] = fz
            return carry

        lax.fori_loop(0, _S + 1, zbody, 0)

        iota = lax.iota(jnp.int32, 16)
        cols = [iota + 16 * j for j in range(D // 16)]
        losplat = jnp.full((16,), lo, jnp.int32)
        dump = jnp.full((16,), _S, jnp.int32)

        def load(g, pbuf, ibuf, sem):
            ge = jnp.minimum(gs + g, _NGR - 1)
            pltpu.async_copy(y_hbm.at[pl.ds(ge * _HG, _HG)], pbuf, sem)
            pltpu.async_copy(idx_hbm.at[pl.ds(base_row + ge * _G, _G)],
                             ibuf.at[pl.ds(0, _G)], sem)

        def wait_load(pbuf, ibuf, sem):
            pltpu.make_async_copy(y_hbm.at[pl.ds(0, _HG)], pbuf, sem).wait()
            pltpu.make_async_copy(idx_hbm.at[pl.ds(0, _G)],
                                  ibuf.at[pl.ds(0, _G)], sem).wait()

        def tgt_of(idv):
            t = idv - losplat
            oob = (t < 0) | (t >= _S)
            return jnp.where(oob, dump, t)

        def process(pbuf, ibuf):
            ivs = [ibuf[pl.ds(16 * m, 16)] for m in range(_G // 16)]
            for m in range(_HG // 16):
                ivlo, ivhi = ivs[m], ivs[m + _HG // 16]

                def lbody(l, carry, m=m, ivlo=ivlo, ivhi=ivhi):
                    wr = 16 * m + l
                    lsplat = jnp.full((16,), l, jnp.int32)
                    t1 = tgt_of(_rgather(ivlo, lsplat))
                    t2 = tgt_of(_rgather(ivhi, lsplat))
                    for j in range(D // 16):
                        v = pbuf[wr, pl.ds(16 * j, 16)]
                        lov = lax.bitcast_convert_type(v << 16, jnp.float32)
                        hiv = lax.bitcast_convert_type(
                            v & _MASK_HI, jnp.float32)
                        plsc.addupdate_scatter(acc, [t1, cols[j]], lov)
                        plsc.addupdate_scatter(acc, [t2, cols[j]], hiv)
                    return carry

                lax.fori_loop(0, 16, lbody, 0)

        @pl.when(cnt > 0)
        def _():
            load(0, pa, ia, sla)

        @pl.when(cnt > 1)
        def _():
            load(1, pb, ib, slb)

        def pair(kk, carry):
            e = 2 * kk
            o = e + 1
            wait_load(pa, ia, sla)
            process(pa, ia)

            @pl.when(e + 2 < cnt)
            def _():
                load(e + 2, pa, ia, sla)

            @pl.when(o < cnt)
            def _():
                wait_load(pb, ib, slb)
                process(pb, ib)

                @pl.when(o + 2 < cnt)
                def _():
                    load(o + 2, pb, ib, slb)

            return carry

        lax.fori_loop(0, (cnt + 1) // 2, pair, 0)

        @pl.when(wid < 31)
        def _():
            pltpu.sync_copy(acc.at[pl.ds(0, _S)],
                            out_hbm.at[pl.ds(wid * _S, _S)])

        @pl.when(wid == 31)
        def _():
            pltpu.sync_copy(acc.at[pl.ds(0, _LAST)],
                            out_hbm.at[pl.ds(31 * _S, _LAST)])

    return _segsum_body


def _make_segsum(k):
    return functools.partial(
        pl.kernel,
        out_type=jax.ShapeDtypeStruct((NSEG, D), jnp.float32),
        mesh=plsc.VectorSubcoreMesh(core_axis_name="c", subcore_axis_name="s"),
        compiler_params=pltpu.CompilerParams(needs_layout_passes=False),
        scratch_types=[
            pltpu.VMEM((2 * _NW,), jnp.int32),
            pltpu.VMEM((_HG, D), jnp.int32),
            pltpu.VMEM((_HG, D), jnp.int32),
            pltpu.VMEM((_G,), jnp.int32),
            pltpu.VMEM((_G,), jnp.int32),
            pltpu.VMEM((_S + 1, D), jnp.float32),
            pltpu.SemaphoreType.DMA,
            pltpu.SemaphoreType.DMA,
        ],
    )(_make_segsum_body(k))


_segsum_calls = [_make_segsum(k) for k in range(_NC)]


_FIN_BLK = 2000


def _fin_body(*refs):
    in_refs, o_ref = refs[:-1], refs[-1]
    acc = in_refs[0][...]
    for r in in_refs[1:]:
        acc = acc + r[...]
    r = jnp.maximum(acc, 0.0)
    m = jnp.max(r, axis=1, keepdims=True)
    o_ref[...] = r / m


def _finalize(partials):
    nblk = NSEG // _FIN_BLK
    return pl.pallas_call(
        _fin_body,
        grid=(nblk,),
        in_specs=[pl.BlockSpec((_FIN_BLK, D), lambda i: (i, 0))
                  for _ in partials],
        out_specs=pl.BlockSpec((_FIN_BLK, D), lambda i: (i, 0)),
        out_shape=jax.ShapeDtypeStruct((NSEG, D), jnp.float32),
    )(*partials)


def kernel(x, batch, W, b):
    batch32 = batch.astype(jnp.int32)
    b2d = b.reshape(1, D)

    edges = jnp.arange(_NW + 1, dtype=jnp.int32) * _S
    ss = jnp.searchsorted(batch32, edges, side="left").astype(jnp.int32)
    st, en = ss[:-1], ss[1:]

    partials = []
    for k in range(_NC):
        y_k = _matmul_relu_chunk(x, W, b2d, k)
        c0, c1 = k * _CHUNK, (k + 1) * _CHUNK
        st_k = jnp.clip(st, c0, c1) - c0
        en_k = jnp.clip(en, c0, c1) - c0
        gs = st_k // _G
        ge = -(-en_k // _G)
        cnt = jnp.where(en_k > st_k, ge - gs, 0)
        bnd = jnp.concatenate([gs, cnt]).astype(jnp.int32)
        partials.append(_segsum_calls[k](y_k, batch32, bnd))
    return _finalize(partials)


# speedup vs baseline: 1.8814x; 1.8814x over previous
import functools

import jax
import jax.numpy as jnp
from jax import lax
from jax.experimental import pallas as pl
from jax.experimental.pallas import tpu as pltpu
from jax.experimental.pallas import tpu_sc as plsc

NSEG = 10000
N = 320000
D = 128

_NC = 4
_CHUNK = N // _NC

_G = 64
_HG = _G // 2
_MASK_HI = -65536
_MASK_LO = 0xFFFF


_MM_BLK = 3200


def _mm_body(x_ref, w_ref, b_ref, y_ref):
    y = lax.dot_general(
        x_ref[...].astype(jnp.bfloat16), w_ref[...].astype(jnp.bfloat16),
        (((1,), (1,)), ((), ())),
        preferred_element_type=jnp.float32)
    y = jnp.maximum(y + b_ref[...], 0.0)
    bits = lax.bitcast_convert_type(
        y.astype(jnp.bfloat16).astype(jnp.float32), jnp.int32)
    z = bits.reshape(_MM_BLK // _G, _G, D)
    a = z[:, :_HG, :]
    b = z[:, _HG:, :]
    w = ((a >> 16) & _MASK_LO) | (b & _MASK_HI)
    y_ref[...] = w.reshape(_MM_BLK // 2, D)


def _matmul_relu_chunk(x, W, b2d, k):
    nblk = _CHUNK // _MM_BLK
    return pl.pallas_call(
        _mm_body,
        grid=(nblk,),
        in_specs=[
            pl.BlockSpec((_MM_BLK, D), lambda i, k=k, nblk=nblk: (i + k * nblk, 0)),
            pl.BlockSpec((D, D), lambda i: (0, 0)),
            pl.BlockSpec((1, D), lambda i: (0, 0)),
        ],
        out_specs=pl.BlockSpec((_MM_BLK // 2, D), lambda i: (i, 0)),
        out_shape=jax.ShapeDtypeStruct((_CHUNK // 2, D), jnp.int32),
    )(x, W, b2d)


_NGR = _CHUNK // _G
_NW = 32
_GPW = _NGR // _NW
_REM = _NGR - _GPW * _NW
_GPAD = -(-(_GPW + 1) // 8) * 8
_FL = 624


def _unpack(src, dst):
    def row_body(r, carry):
        for j in range(D // 16):
            v = src[r, pl.ds(16 * j, 16)]
            lo = lax.bitcast_convert_type(v << 16, jnp.float32)
            hi = lax.bitcast_convert_type(v & _MASK_HI, jnp.float32)
            dst[r, pl.ds(16 * j, 16)] = lo
            dst[r + _HG, pl.ds(16 * j, 16)] = hi
        return carry
    lax.fori_loop(0, _HG, row_body, 0)


def _segsum_body(y_hbm, idx_hbm, zeros_hbm, out_hbm,
                 idx_v, pa, pb, fa, fb, acc, sla, slb, ssa, ssb):
    c = lax.axis_index("c")
    s = lax.axis_index("s")
    wid = c * 16 + s
    g0 = wid * _GPW + jnp.minimum(wid, _REM)
    has_extra = wid < _REM

    def load(g, buf, sem):
        return pltpu.async_copy(y_hbm.at[pl.ds((g0 + g) * _HG, _HG)], buf, sem)

    def wait_load(buf, sem):
        pltpu.make_async_copy(y_hbm.at[pl.ds(0, _HG)], buf, sem).wait()

    def scat(g, buf, sem):
        return pltpu.async_copy(buf, acc.at[idx_v.at[g]], sem, add=True)

    def wait_scat(g, buf, sem):
        pltpu.make_async_copy(buf, acc.at[idx_v.at[g]], sem).wait()

    pltpu.sync_copy(zeros_hbm, acc.at[pl.ds(s * _FL, _FL)])

    @pl.when(s == 15)
    def _():
        pltpu.sync_copy(zeros_hbm.at[pl.ds(0, 16)],
                        acc.at[pl.ds(16 * _FL, 16)])

    pltpu.sync_copy(idx_hbm.at[wid], idx_v)

    plsc.subcore_barrier()

    load(0, pa, sla)
    load(1, pb, slb)

    def body(k, carry):
        e = 2 * k
        o = 2 * k + 1
        wait_load(pa, sla)

        @pl.when(k > 0)
        def _():
            wait_scat(jnp.maximum(e - 2, 0), fa, ssa)

        _unpack(pa, fa)

        @pl.when(e + 2 < _GPW)
        def _():
            load(e + 2, pa, sla)

        scat(e, fa, ssa)

        wait_load(pb, slb)

        @pl.when(k > 0)
        def _():
            wait_scat(jnp.maximum(o - 2, 1), fb, ssb)

        _unpack(pb, fb)

        @pl.when(o + 2 < _GPW)
        def _():
            load(o + 2, pb, slb)

        scat(o, fb, ssb)
        return carry

    lax.fori_loop(0, _GPW // 2, body, 0)

    if _GPW % 2 == 1:
        wait_load(pa, sla)
        wait_scat(_GPW - 3, fa, ssa)
        _unpack(pa, fa)
        pltpu.sync_copy(fa, acc.at[idx_v.at[_GPW - 1]], add=True)
        wait_scat(_GPW - 2, fb, ssb)
    else:
        wait_scat(_GPW - 2, fa, ssa)
        wait_scat(_GPW - 1, fb, ssb)

    @pl.when(has_extra)
    def _():
        pltpu.sync_copy(y_hbm.at[pl.ds((g0 + _GPW) * _HG, _HG)], pb)
        _unpack(pb, fb)
        pltpu.sync_copy(fb, acc.at[idx_v.at[_GPW]], add=True)

    plsc.subcore_barrier()

    pltpu.sync_copy(acc.at[pl.ds(s * _FL, _FL)],
                    out_hbm.at[pl.ds(c * NSEG + s * _FL, _FL)])

    @pl.when(s == 15)
    def _():
        pltpu.sync_copy(acc.at[pl.ds(16 * _FL, 16)],
                        out_hbm.at[pl.ds(c * NSEG + 16 * _FL, 16)])


_segsum = functools.partial(
    pl.kernel,
    out_type=jax.ShapeDtypeStruct((2 * NSEG, D), jnp.float32),
    mesh=plsc.VectorSubcoreMesh(core_axis_name="c", subcore_axis_name="s"),
    scratch_types=[
        pltpu.VMEM((_GPAD, _G), jnp.int32),
        pltpu.VMEM((_HG, D), jnp.int32),
        pltpu.VMEM((_HG, D), jnp.int32),
        pltpu.VMEM((_G, D), jnp.float32),
        pltpu.VMEM((_G, D), jnp.float32),
        pltpu.VMEM_SHARED((NSEG, D), jnp.float32),
        pltpu.SemaphoreType.DMA,
        pltpu.SemaphoreType.DMA,
        pltpu.SemaphoreType.DMA,
        pltpu.SemaphoreType.DMA,
    ],
)(_segsum_body)


_FIN_BLK = 2000


def _fin_body(*refs):
    in_refs, o_ref = refs[:-1], refs[-1]
    acc = in_refs[0][...]
    for r in in_refs[1:]:
        acc = acc + r[...]
    r = jnp.maximum(acc, 0.0)
    m = jnp.max(r, axis=1, keepdims=True)
    o_ref[...] = r / m


def _finalize(partials):
    nblk = NSEG // _FIN_BLK
    in_specs = []
    args = []
    for p in partials:
        in_specs.append(pl.BlockSpec((_FIN_BLK, D), lambda i: (i, 0)))
        in_specs.append(
            pl.BlockSpec((_FIN_BLK, D), lambda i, nblk=nblk: (i + nblk, 0)))
        args += [p, p]
    return pl.pallas_call(
        _fin_body,
        grid=(nblk,),
        in_specs=in_specs,
        out_specs=pl.BlockSpec((_FIN_BLK, D), lambda i: (i, 0)),
        out_shape=jax.ShapeDtypeStruct((NSEG, D), jnp.float32),
    )(*args)


def kernel(x, batch, W, b):
    idx2d = batch.astype(jnp.int32).reshape(N // _G, _G)
    idx2d_pad = jnp.concatenate(
        [idx2d, jnp.zeros((_GPAD, _G), jnp.int32)], axis=0)
    zeros = jnp.zeros((_FL, D), jnp.float32)
    b2d = b.reshape(1, D)

    partials = []
    for k in range(_NC):
        y_k = _matmul_relu_chunk(x, W, b2d, k)
        base = k * _NGR
        idx_w = jnp.stack([
            lax.dynamic_slice_in_dim(
                idx2d_pad, base + w * _GPW + min(w, _REM), _GPAD)
            for w in range(_NW)
        ])
        partials.append(_segsum(y_k, idx_w, zeros))
    return _finalize(partials)


# speedup vs baseline: 2.3474x; 1.2477x over previous
import functools

import jax
import jax.numpy as jnp
from jax import lax
from jax.experimental import pallas as pl
from jax.experimental.pallas import tpu as pltpu
from jax.experimental.pallas import tpu_sc as plsc

NSEG = 10000
N = 320000
D = 128

_NC = 2
_CHUNK = N // _NC

_G = 64
_HG = _G // 2
_MASK_HI = -65536
_MASK_LO = 0xFFFF


_MM_BLK = 6400


def _mm_body(x_ref, w_ref, b_ref, y_ref):
    y = lax.dot_general(
        x_ref[...].astype(jnp.bfloat16), w_ref[...].astype(jnp.bfloat16),
        (((1,), (1,)), ((), ())),
        preferred_element_type=jnp.float32)
    y = jnp.maximum(y + b_ref[...], 0.0)
    bits = lax.bitcast_convert_type(
        y.astype(jnp.bfloat16).astype(jnp.float32), jnp.int32)
    z = bits.reshape(_MM_BLK // _G, _G, D)
    a = z[:, :_HG, :]
    b = z[:, _HG:, :]
    w = ((a >> 16) & _MASK_LO) | (b & _MASK_HI)
    y_ref[...] = w.reshape(_MM_BLK // 2, D)


def _matmul_relu_chunk(x, W, b2d, k):
    nblk = _CHUNK // _MM_BLK
    return pl.pallas_call(
        _mm_body,
        grid=(nblk,),
        in_specs=[
            pl.BlockSpec((_MM_BLK, D), lambda i, k=k, nblk=nblk: (i + k * nblk, 0)),
            pl.BlockSpec((D, D), lambda i: (0, 0)),
            pl.BlockSpec((1, D), lambda i: (0, 0)),
        ],
        out_specs=pl.BlockSpec((_MM_BLK // 2, D), lambda i: (i, 0)),
        out_shape=jax.ShapeDtypeStruct((_CHUNK // 2, D), jnp.int32),
    )(x, W, b2d)


_NGR = _CHUNK // _G
_NW = 32
_GPW = _NGR // _NW
_REM = _NGR - _GPW * _NW
_GPAD = -(-(_GPW + 1) // 8) * 8
_FL = 624

assert _GPW % 2 == 0


def _unpack(src, dst):
    def row_body(r, carry):
        for j in range(D // 16):
            v = src[r, pl.ds(16 * j, 16)]
            lo = lax.bitcast_convert_type(v << 16, jnp.float32)
            hi = lax.bitcast_convert_type(v & _MASK_HI, jnp.float32)
            dst[r, pl.ds(16 * j, 16)] = lo
            dst[r + _HG, pl.ds(16 * j, 16)] = hi
        return carry
    lax.fori_loop(0, _HG, row_body, 0)


def _segsum_body(y_hbm, idx_hbm, zeros_hbm, out_hbm,
                 idx_v, pa, pb, fa, fb, acc, sla, slb, ssa, ssb):
    c = lax.axis_index("c")
    s = lax.axis_index("s")
    wid = c * 16 + s
    g0 = wid * _GPW + jnp.minimum(wid, _REM)
    has_extra = wid < _REM

    def load(g, buf, sem):
        return pltpu.async_copy(y_hbm.at[pl.ds((g0 + g) * _HG, _HG)], buf, sem)

    def wait_load(buf, sem):
        pltpu.make_async_copy(y_hbm.at[pl.ds(0, _HG)], buf, sem).wait()

    def scat(g, buf, sem):
        return pltpu.async_copy(buf, acc.at[idx_v.at[g]], sem, add=True)

    def wait_scat(g, buf, sem):
        pltpu.make_async_copy(buf, acc.at[idx_v.at[g]], sem).wait()

    pltpu.sync_copy(zeros_hbm, acc.at[pl.ds(s * _FL, _FL)])

    @pl.when(s == 15)
    def _():
        pltpu.sync_copy(zeros_hbm.at[pl.ds(0, 16)],
                        acc.at[pl.ds(16 * _FL, 16)])

    pltpu.sync_copy(idx_hbm.at[wid], idx_v)

    plsc.subcore_barrier()

    load(0, pa, sla)
    load(1, pb, slb)

    def body(k, carry):
        e = 2 * k
        o = 2 * k + 1
        wait_load(pa, sla)

        @pl.when(k > 0)
        def _():
            wait_scat(jnp.maximum(e - 2, 0), fa, ssa)

        _unpack(pa, fa)

        @pl.when(e + 2 < _GPW)
        def _():
            load(e + 2, pa, sla)

        scat(e, fa, ssa)

        wait_load(pb, slb)

        @pl.when(k > 0)
        def _():
            wait_scat(jnp.maximum(o - 2, 1), fb, ssb)

        _unpack(pb, fb)

        @pl.when(o + 2 < _GPW)
        def _():
            load(o + 2, pb, slb)

        scat(o, fb, ssb)
        return carry

    lax.fori_loop(0, _GPW // 2, body, 0)

    wait_scat(_GPW - 2, fa, ssa)
    wait_scat(_GPW - 1, fb, ssb)

    @pl.when(has_extra)
    def _():
        pltpu.sync_copy(y_hbm.at[pl.ds((g0 + _GPW) * _HG, _HG)], pb)
        _unpack(pb, fb)
        pltpu.sync_copy(fb, acc.at[idx_v.at[_GPW]], add=True)

    plsc.subcore_barrier()

    pltpu.sync_copy(acc.at[pl.ds(s * _FL, _FL)],
                    out_hbm.at[pl.ds(c * NSEG + s * _FL, _FL)])

    @pl.when(s == 15)
    def _():
        pltpu.sync_copy(acc.at[pl.ds(16 * _FL, 16)],
                        out_hbm.at[pl.ds(c * NSEG + 16 * _FL, 16)])


_segsum = functools.partial(
    pl.kernel,
    out_type=jax.ShapeDtypeStruct((2 * NSEG, D), jnp.float32),
    mesh=plsc.VectorSubcoreMesh(core_axis_name="c", subcore_axis_name="s"),
    scratch_types=[
        pltpu.VMEM((_GPAD, _G), jnp.int32),
        pltpu.VMEM((_HG, D), jnp.int32),
        pltpu.VMEM((_HG, D), jnp.int32),
        pltpu.VMEM((_G, D), jnp.float32),
        pltpu.VMEM((_G, D), jnp.float32),
        pltpu.VMEM_SHARED((NSEG, D), jnp.float32),
        pltpu.SemaphoreType.DMA,
        pltpu.SemaphoreType.DMA,
        pltpu.SemaphoreType.DMA,
        pltpu.SemaphoreType.DMA,
    ],
)(_segsum_body)


_FIN_BLK = 2000


def _fin_body(*refs):
    in_refs, o_ref = refs[:-1], refs[-1]
    acc = in_refs[0][...]
    for r in in_refs[1:]:
        acc = acc + r[...]
    r = jnp.maximum(acc, 0.0)
    m = jnp.max(r, axis=1, keepdims=True)
    o_ref[...] = r / m


def _finalize(partials):
    nblk = NSEG // _FIN_BLK
    in_specs = []
    args = []
    for p in partials:
        in_specs.append(pl.BlockSpec((_FIN_BLK, D), lambda i: (i, 0)))
        in_specs.append(
            pl.BlockSpec((_FIN_BLK, D), lambda i, nblk=nblk: (i + nblk, 0)))
        args += [p, p]
    return pl.pallas_call(
        _fin_body,
        grid=(nblk,),
        in_specs=in_specs,
        out_specs=pl.BlockSpec((_FIN_BLK, D), lambda i: (i, 0)),
        out_shape=jax.ShapeDtypeStruct((NSEG, D), jnp.float32),
    )(*args)


def kernel(x, batch, W, b):
    idx2d = batch.astype(jnp.int32).reshape(N // _G, _G)
    idx2d_pad = jnp.concatenate(
        [idx2d, jnp.zeros((_GPAD, _G), jnp.int32)], axis=0)
    zeros = jnp.zeros((_FL, D), jnp.float32)
    b2d = b.reshape(1, D)

    partials = []
    for k in range(_NC):
        y_k = _matmul_relu_chunk(x, W, b2d, k)
        base = k * _NGR
        idx_w = jnp.stack([
            lax.dynamic_slice_in_dim(
                idx2d_pad, base + w * _GPW + min(w, _REM), _GPAD)
            for w in range(_NW)
        ])
        partials.append(_segsum(y_k, idx_w, zeros))
    return _finalize(partials)


# speedup vs baseline: 2.3804x; 1.0141x over previous
import functools

import jax
import jax.numpy as jnp
from jax import lax
from jax.experimental import pallas as pl
from jax.experimental.pallas import tpu as pltpu
from jax.experimental.pallas import tpu_sc as plsc

NSEG = 10000
N = 320000
D = 128

_NC = 2
_CHUNK = N // _NC

_G = 64
_HG = _G // 2
_MASK_HI = -65536
_MASK_LO = 0xFFFF


_MM_BLK = 16000


def _mm_body(x_ref, w_ref, b_ref, y_ref):
    y = lax.dot_general(
        x_ref[...].astype(jnp.bfloat16), w_ref[...].astype(jnp.bfloat16),
        (((1,), (1,)), ((), ())),
        preferred_element_type=jnp.float32)
    y = jnp.maximum(y + b_ref[...], 0.0)
    bits = lax.bitcast_convert_type(
        y.astype(jnp.bfloat16).astype(jnp.float32), jnp.int32)
    z = bits.reshape(_MM_BLK // _G, _G, D)
    a = z[:, :_HG, :]
    b = z[:, _HG:, :]
    w = ((a >> 16) & _MASK_LO) | (b & _MASK_HI)
    y_ref[...] = w.reshape(_MM_BLK // 2, D)


def _matmul_relu_chunk(x, W, b2d, k):
    nblk = _CHUNK // _MM_BLK
    return pl.pallas_call(
        _mm_body,
        grid=(nblk,),
        in_specs=[
            pl.BlockSpec((_MM_BLK, D), lambda i, k=k, nblk=nblk: (i + k * nblk, 0)),
            pl.BlockSpec((D, D), lambda i: (0, 0)),
            pl.BlockSpec((1, D), lambda i: (0, 0)),
        ],
        out_specs=pl.BlockSpec((_MM_BLK // 2, D), lambda i: (i, 0)),
        out_shape=jax.ShapeDtypeStruct((_CHUNK // 2, D), jnp.int32),
    )(x, W, b2d)


_NGR = _CHUNK // _G
_NW = 32
_GPW = _NGR // _NW
_REM = _NGR - _GPW * _NW
_GPAD = -(-(_GPW + 1) // 8) * 8
_FL = 624

assert _GPW % 2 == 0


def _unpack(src, dst):
    def row_body(r, carry):
        for j in range(D // 16):
            v = src[r, pl.ds(16 * j, 16)]
            lo = lax.bitcast_convert_type(v << 16, jnp.float32)
            hi = lax.bitcast_convert_type(v & _MASK_HI, jnp.float32)
            dst[r, pl.ds(16 * j, 16)] = lo
            dst[r + _HG, pl.ds(16 * j, 16)] = hi
        return carry
    lax.fori_loop(0, _HG, row_body, 0)


def _segsum_body(y_hbm, idx_hbm, zeros_hbm, out_hbm,
                 idx_v, pa, pb, fa, fb, acc, sla, slb, ssa, ssb):
    c = lax.axis_index("c")
    s = lax.axis_index("s")
    wid = c * 16 + s
    g0 = wid * _GPW + jnp.minimum(wid, _REM)
    has_extra = wid < _REM

    def load(g, buf, sem):
        return pltpu.async_copy(y_hbm.at[pl.ds((g0 + g) * _HG, _HG)], buf, sem)

    def wait_load(buf, sem):
        pltpu.make_async_copy(y_hbm.at[pl.ds(0, _HG)], buf, sem).wait()

    def scat(g, buf, sem):
        return pltpu.async_copy(buf, acc.at[idx_v.at[g]], sem, add=True)

    def wait_scat(g, buf, sem):
        pltpu.make_async_copy(buf, acc.at[idx_v.at[g]], sem).wait()

    pltpu.sync_copy(zeros_hbm, acc.at[pl.ds(s * _FL, _FL)])

    @pl.when(s == 15)
    def _():
        pltpu.sync_copy(zeros_hbm.at[pl.ds(0, 16)],
                        acc.at[pl.ds(16 * _FL, 16)])

    pltpu.sync_copy(idx_hbm.at[wid], idx_v)

    plsc.subcore_barrier()

    load(0, pa, sla)
    load(1, pb, slb)

    def body(k, carry):
        e = 2 * k
        o = 2 * k + 1
        wait_load(pa, sla)

        @pl.when(k > 0)
        def _():
            wait_scat(jnp.maximum(e - 2, 0), fa, ssa)

        _unpack(pa, fa)

        @pl.when(e + 2 < _GPW)
        def _():
            load(e + 2, pa, sla)

        scat(e, fa, ssa)

        wait_load(pb, slb)

        @pl.when(k > 0)
        def _():
            wait_scat(jnp.maximum(o - 2, 1), fb, ssb)

        _unpack(pb, fb)

        @pl.when(o + 2 < _GPW)
        def _():
            load(o + 2, pb, slb)

        scat(o, fb, ssb)
        return carry

    lax.fori_loop(0, _GPW // 2, body, 0)

    wait_scat(_GPW - 2, fa, ssa)
    wait_scat(_GPW - 1, fb, ssb)

    @pl.when(has_extra)
    def _():
        pltpu.sync_copy(y_hbm.at[pl.ds((g0 + _GPW) * _HG, _HG)], pb)
        _unpack(pb, fb)
        pltpu.sync_copy(fb, acc.at[idx_v.at[_GPW]], add=True)

    plsc.subcore_barrier()

    pltpu.sync_copy(acc.at[pl.ds(s * _FL, _FL)],
                    out_hbm.at[pl.ds(c * NSEG + s * _FL, _FL)])

    @pl.when(s == 15)
    def _():
        pltpu.sync_copy(acc.at[pl.ds(16 * _FL, 16)],
                        out_hbm.at[pl.ds(c * NSEG + 16 * _FL, 16)])


_segsum = functools.partial(
    pl.kernel,
    out_type=jax.ShapeDtypeStruct((2 * NSEG, D), jnp.float32),
    mesh=plsc.VectorSubcoreMesh(core_axis_name="c", subcore_axis_name="s"),
    scratch_types=[
        pltpu.VMEM((_GPAD, _G), jnp.int32),
        pltpu.VMEM((_HG, D), jnp.int32),
        pltpu.VMEM((_HG, D), jnp.int32),
        pltpu.VMEM((_G, D), jnp.float32),
        pltpu.VMEM((_G, D), jnp.float32),
        pltpu.VMEM_SHARED((NSEG, D), jnp.float32),
        pltpu.SemaphoreType.DMA,
        pltpu.SemaphoreType.DMA,
        pltpu.SemaphoreType.DMA,
        pltpu.SemaphoreType.DMA,
    ],
)(_segsum_body)


_FIN_BLK = 2000


def _fin_body(*refs):
    in_refs, o_ref = refs[:-1], refs[-1]
    acc = in_refs[0][...]
    for r in in_refs[1:]:
        acc = acc + r[...]
    r = jnp.maximum(acc, 0.0)
    m = jnp.max(r, axis=1, keepdims=True)
    o_ref[...] = r / m


def _finalize(partials):
    nblk = NSEG // _FIN_BLK
    in_specs = []
    args = []
    for p in partials:
        in_specs.append(pl.BlockSpec((_FIN_BLK, D), lambda i: (i, 0)))
        in_specs.append(
            pl.BlockSpec((_FIN_BLK, D), lambda i, nblk=nblk: (i + nblk, 0)))
        args += [p, p]
    return pl.pallas_call(
        _fin_body,
        grid=(nblk,),
        in_specs=in_specs,
        out_specs=pl.BlockSpec((_FIN_BLK, D), lambda i: (i, 0)),
        out_shape=jax.ShapeDtypeStruct((NSEG, D), jnp.float32),
    )(*args)


def kernel(x, batch, W, b):
    idx2d = batch.astype(jnp.int32).reshape(N // _G, _G)
    idx2d_pad = jnp.concatenate(
        [idx2d, jnp.zeros((_GPAD, _G), jnp.int32)], axis=0)
    zeros = jnp.zeros((_FL, D), jnp.float32)
    b2d = b.reshape(1, D)

    partials = []
    for k in range(_NC):
        y_k = _matmul_relu_chunk(x, W, b2d, k)
        base = k * _NGR
        idx_w = jnp.stack([
            lax.dynamic_slice_in_dim(
                idx2d_pad, base + w * _GPW + min(w, _REM), _GPAD)
            for w in range(_NW)
        ])
        partials.append(_segsum(y_k, idx_w, zeros))
    return _finalize(partials)
